# baseline (speedup 1.0000x reference)
import functools

import jax
import jax.numpy as jnp
from jax import lax
from jax.experimental import pallas as pl
from jax.experimental.pallas import tpu as pltpu
from jax.experimental.pallas import tpu_sc as plsc

NC = 2
NS = 16
L = 16
NW = NC * NS
DIM = 64
ND = DIM // L
ROWS_W = 1568
N_PAD = ROWS_W * NW
C = 128
UNROLL = 8

_params = pltpu.CompilerParams(needs_layout_passes=False,
                               use_tc_tiling_on_sc=False)


def _worker_id():
    return lax.axis_index("s") * NC + lax.axis_index("c")


def _splat(ref, idx):
    return jnp.max(plsc.load_gather(ref, [jnp.full((L,), idx, jnp.int32)]))


def _make_compact(e_pad, nblk):
    mesh = plsc.VectorSubcoreMesh(core_axis_name="c", subcore_axis_name="s")

    @functools.partial(
        pl.kernel,
        mesh=mesh,
        compiler_params=_params,
        out_type=(jax.ShapeDtypeStruct((nblk, 3, C), jnp.int32),
                  jax.ShapeDtypeStruct((NW * L,), jnp.int32)),
        scratch_types=[
            pltpu.VMEM((64,), jnp.int32),
            pltpu.VMEM((C,), jnp.int32),
            pltpu.VMEM((C,), jnp.int32),
            pltpu.VMEM((C,), jnp.int32),
            pltpu.VMEM((C,), jnp.int32),
            pltpu.VMEM((C,), jnp.float32),
            pltpu.VMEM((C,), jnp.float32),
            pltpu.VMEM((2 * C,), jnp.int32),
            pltpu.VMEM((2 * C,), jnp.int32),
            pltpu.VMEM((2 * C,), jnp.int32),
            pltpu.VMEM((L,), jnp.int32),
            pltpu.SemaphoreType.DMA,
            pltpu.SemaphoreType.DMA,
        ],
    )
    def compact(row_hbm, col_hbm, val_hbm, starts_hbm, packed_hbm, cnt_hbm,
                starts_v, col_a, col_b, row_a, row_b, val_a, val_b,
                scol, sbase, sval, cnts, esem_a, esem_b):
        wid = _worker_id()
        r0 = wid * ROWS_W

        pltpu.sync_copy(starts_hbm, starts_v)
        start = _splat(starts_v, wid)
        end = _splat(starts_v, wid + 1)
        start_al = jnp.bitwise_and(start, jnp.int32(-8))
        nchunks = jnp.maximum((end - start_al + (C - 1)) // C, 1)
        npairs = (nchunks + 1) // 2
        b0 = start // C + 4 * wid

        bufs = ((col_a, row_a, val_a, esem_a),
                (col_b, row_b, val_b, esem_b))

        def _issue_edges(b, e):
            col, row, val, esem = bufs[b]
            pltpu.async_copy(col_hbm.at[pl.ds(e, C)], col, esem)
            pltpu.async_copy(row_hbm.at[pl.ds(e, C)], row, esem)
            pltpu.async_copy(val_hbm.at[pl.ds(e, C)], val, esem)

        def _wait_edges(b, e):
            col, row, val, esem = bufs[b]
            pltpu.make_async_copy(col_hbm.at[pl.ds(e, C)], col, esem).wait()
            pltpu.make_async_copy(row_hbm.at[pl.ds(e, C)], row, esem).wait()
            pltpu.make_async_copy(val_hbm.at[pl.ds(e, C)], val, esem).wait()

        def _flush(from_off, blk):
            pltpu.sync_copy(scol.at[pl.ds(from_off, C)],
                            packed_hbm.at[blk, 0])
            pltpu.sync_copy(sbase.at[pl.ds(from_off, C)],
                            packed_hbm.at[blk, 1])
            pltpu.sync_copy(sval.at[pl.ds(from_off, C)],
                            packed_hbm.at[blk, 2])

        _issue_edges(0, pl.multiple_of(start_al, 8))
        _issue_edges(1, pl.multiple_of(start_al + C, 8))

        def _pair(kk, carry):
            for b in (0, 1):
                k = kk * 2 + b
                col, row, val, _ = bufs[b]
                off, nb = carry
                _wait_edges(b, pl.multiple_of(start_al + k * C, 8))

                for i in range(C // L):
                    sl = pl.ds(i * L, L)
                    rl = row[sl] - r0
                    v = val[sl]
                    ok = (rl >= 0) & (rl < ROWS_W) & (v != 0.0)
                    basev = jnp.clip(rl, 0, ROWS_W - 1) * DIM
                    plsc.store_compressed(scol.at[pl.ds(off, L)],
                                          col[sl], mask=ok)
                    plsc.store_compressed(sbase.at[pl.ds(off, L)],
                                          basev, mask=ok)
                    plsc.store_compressed(sval.at[pl.ds(off, L)],
                                          plsc.bitcast(v, jnp.int32),
                                          mask=ok)
                    off = off + jnp.max(
                        plsc.all_reduce_population_count(ok))

                _issue_edges(b, pl.multiple_of(start_al + (k + 2) * C, 8))

                do_flush = off >= C

                @pl.when(do_flush)
                def _():
                    _flush(0, b0 + nb)
                    for i in range(C // L):
                        scol[pl.ds(i * L, L)] = scol[pl.ds(C + i * L, L)]
                        sbase[pl.ds(i * L, L)] = sbase[pl.ds(C + i * L, L)]
                        sval[pl.ds(i * L, L)] = sval[pl.ds(C + i * L, L)]

                off = jnp.where(do_flush, off - C, off)
                nb = jnp.where(do_flush, nb + 1, nb)
                carry = (off, nb)
            return carry

        off, nb = lax.fori_loop(0, npairs, _pair,
                                (jnp.int32(0), jnp.int32(0)))

        last = 2 * npairs
        _wait_edges(0, pl.multiple_of(start_al + last * C, 8))
        _wait_edges(1, pl.multiple_of(start_al + (last + 1) * C, 8))

        lane0 = lax.broadcasted_iota(jnp.int32, (L,), 0)
        zi = jnp.zeros((L,), jnp.int32)
        for i in range(2 * C // L):
            sl = pl.ds(i * L, L)
            m = (i * L + lane0) < off
            scol[sl] = jnp.where(m, scol[sl], zi)
            sbase[sl] = jnp.where(m, sbase[sl], zi)
            sval[sl] = jnp.where(m, sval[sl], zi)

        has_tail = off > 0

        @pl.when(has_tail)
        def _():
            _flush(0, b0 + nb)

        nb = jnp.where(has_tail, nb + 1, nb)
        odd = jnp.bitwise_and(nb, 1) == 1

        @pl.when(odd)
        def _():
            _flush(C, b0 + nb)

        nb = jnp.where(odd, nb + 1, nb)
        empty = nb == 0

        @pl.when(empty)
        def _():
            _flush(C, b0)
            _flush(C, b0 + 1)

        nb = jnp.where(empty, 2, nb)
        cnts[pl.ds(0, L)] = jnp.full((L,), 0, jnp.int32) + nb
        pltpu.sync_copy(cnts, cnt_hbm.at[pl.ds(wid * L, L)])

    return compact


def _make_layer(nblk):
    mesh = plsc.VectorSubcoreMesh(core_axis_name="c", subcore_axis_name="s")

    @functools.partial(
        pl.kernel,
        mesh=mesh,
        compiler_params=_params,
        out_type=jax.ShapeDtypeStruct((N_PAD * DIM,), jnp.float32),
        scratch_types=[
            pltpu.VMEM((64,), jnp.int32),
            pltpu.VMEM((NW * L,), jnp.int32),
            pltpu.VMEM((3, C), jnp.int32),
            pltpu.VMEM((3, C), jnp.int32),
            pltpu.VMEM((C, DIM // 2), jnp.int32),
            pltpu.VMEM((C, DIM // 2), jnp.int32),
            pltpu.VMEM((ROWS_W * DIM,), jnp.float32),
            pltpu.SemaphoreType.DMA,
            pltpu.SemaphoreType.DMA,
            pltpu.SemaphoreType.DMA,
            pltpu.SemaphoreType.DMA,
        ],
    )
    def layer(ego_hbm, packed_hbm, starts_hbm, cnt_hbm, out_hbm,
              starts_v, cnt_v, eb_a, eb_b, gath_a, gath_b, acc,
              esem_a, esem_b, gsem_a, gsem_b):
        wid = _worker_id()
        r0 = wid * ROWS_W

        pltpu.sync_copy(starts_hbm, starts_v)
        pltpu.sync_copy(cnt_hbm, cnt_v)
        start = _splat(starts_v, wid)
        b0 = start // C + 4 * wid
        nb = _splat(cnt_v, wid * L)
        npairs = nb // 2

        bufs = ((eb_a, gath_a, esem_a, gsem_a),
                (eb_b, gath_b, esem_b, gsem_b))

        def _issue_blk(b, k):
            eb, _, esem, _ = bufs[b]
            pltpu.async_copy(packed_hbm.at[b0 + k], eb, esem)

        def _wait_blk(b, k):
            eb, _, esem, _ = bufs[b]
            pltpu.make_async_copy(packed_hbm.at[b0 + k], eb, esem).wait()

        def _issue_gather(b):
            eb, gath, _, gsem = bufs[b]
            pltpu.async_copy(ego_hbm.at[eb.at[0]], gath, gsem)

        def _wait_gather(b):
            eb, gath, _, gsem = bufs[b]
            pltpu.make_async_copy(ego_hbm.at[eb.at[0]], gath, gsem).wait()

        _issue_blk(0, 0)
        _issue_blk(1, 1)

        zero = jnp.zeros((L,), jnp.float32)

        def _zero_rows(r, carry):
            for u in range(UNROLL):
                acc[pl.ds((r * UNROLL + u) * L, L)] = zero
            return carry

        lax.fori_loop(0, ROWS_W * DIM // (L * UNROLL), _zero_rows, 0)

        _wait_blk(0, 0)
        _issue_gather(0)

        even_iota = [d * 2 * L + 2 * lax.broadcasted_iota(jnp.int32, (L,), 0)
                     for d in range(DIM // (2 * L))]
        one_v = jnp.full((L,), 1, jnp.int32)
        two_v = jnp.full((L,), 2, jnp.int32)

        def _pair(kk, carry):
            for b in (0, 1):
                k = kk * 2 + b
                eb, gath = bufs[b][0], bufs[b][1]
                nbuf = 1 - b

                _wait_gather(b)

                @pl.when(k + 1 < nb)
                def _():
                    _wait_blk(nbuf, k + 1)
                    _issue_gather(nbuf)

                @plsc.parallel_loop(0, C, 1, unroll=UNROLL)
                def _edges(c):
                    cv = jnp.full((L,), c, jnp.int32)
                    bv = plsc.load_gather(eb, [one_v, cv])
                    vv = plsc.bitcast(plsc.load_gather(eb, [two_v, cv]),
                                      jnp.float32)
                    for d in range(DIM // (2 * L)):
                        w = gath[c, pl.ds(d * L, L)]
                        fe = plsc.bitcast(jnp.left_shift(w, 16),
                                          jnp.float32)
                        fo = plsc.bitcast(
                            jnp.bitwise_and(w, jnp.int32(-65536)),
                            jnp.float32)
                        ide = bv + even_iota[d]
                        plsc.addupdate_scatter(acc, [ide], fe * vv)
                        plsc.addupdate_scatter(acc, [ide + 1], fo * vv)

                _issue_blk(b, k + 2)
            return carry

        lax.fori_loop(0, npairs, _pair, 0)

        _wait_blk(0, nb)
        _wait_blk(1, nb + 1)

        pltpu.sync_copy(acc, out_hbm.at[pl.ds(r0 * DIM, ROWS_W * DIM)])

    return layer


def _make_final(batch):
    bpw = batch // NW
    mesh = plsc.VectorSubcoreMesh(core_axis_name="c", subcore_axis_name="s")
    out_sds = jax.ShapeDtypeStruct((batch, DIM), jnp.float32)

    @functools.partial(
        pl.kernel,
        mesh=mesh,
        compiler_params=_params,
        out_type=(out_sds, out_sds),
        scratch_types=[
            pltpu.VMEM((bpw,), jnp.int32),
            pltpu.VMEM((bpw, DIM), jnp.float32),
            pltpu.VMEM((bpw, DIM), jnp.float32),
            pltpu.VMEM((bpw, DIM), jnp.float32),
            pltpu.VMEM((bpw, DIM), jnp.float32),
            pltpu.VMEM((bpw, DIM), jnp.float32),
            pltpu.SemaphoreType.DMA,
        ],
    )
    def final(e0, e1, e2, e3, uidx_hbm, iidx_hbm, uout_hbm, iout_hbm,
              idxv, g0, g1, g2, g3, obuf, sem):
        wid = _worker_id()
        b0 = wid * bpw
        quarter = jnp.float32(0.25)
        for idx_hbm, out_hbm in ((uidx_hbm, uout_hbm), (iidx_hbm, iout_hbm)):
            pltpu.sync_copy(idx_hbm.at[pl.ds(b0, bpw)], idxv)
            for tab, gb in ((e0, g0), (e1, g1), (e2, g2), (e3, g3)):
                pltpu.async_copy(tab.at[idxv], gb, sem).wait()

            def _avg(r, carry):
                for d in range(ND):
                    sl = pl.ds(d * L, L)
                    obuf[r, sl] = (g0[r, sl] + g1[r, sl] + g2[r, sl]
                                   + g3[r, sl]) * quarter
                return carry

            lax.fori_loop(0, bpw, _avg, 0)
            pltpu.sync_copy(obuf, out_hbm.at[pl.ds(b0, bpw)])

    return final


def kernel(users, items, adj_row, adj_col, adj_val, user_emb, item_emb):
    nu, dim = user_emb.shape
    ni = item_emb.shape[0]
    n = nu + ni
    assert dim == DIM and n <= N_PAD
    e_cnt = adj_row.shape[0]

    mkey = jax.random.key(42)
    random_tensor = 0.5 + jax.random.uniform(mkey, adj_val.shape)
    mask = jnp.floor(random_tensor).astype(bool)
    vals = jnp.where(mask, adj_val, 0.0) * 2.0

    adj_row = adj_row.astype(jnp.int32)
    adj_col = adj_col.astype(jnp.int32)

    e_pad = ((e_cnt + C - 1) // C + 4) * C
    pe = e_pad - e_cnt
    rows_p = jnp.pad(adj_row, (0, pe))
    cols_p = jnp.pad(adj_col, (0, pe))
    vals_p = jnp.pad(vals, (0, pe))

    bounds = jnp.arange(NW + 1, dtype=jnp.int32) * ROWS_W
    starts = jnp.searchsorted(adj_row, bounds, side="left").astype(jnp.int32)
    starts64 = jnp.zeros((64,), jnp.int32).at[: NW + 1].set(starts)

    ego0 = jnp.pad(jnp.concatenate([user_emb, item_emb], axis=0),
                   ((0, N_PAD - n), (0, 0)))

    nblk = e_pad // C + 4 * NW + 8
    compact = _make_compact(e_pad, nblk)
    packed, cnt = compact(rows_p, cols_p, vals_p, starts64)

    def _pack_bf16(tbl):
        b = jax.lax.bitcast_convert_type(tbl, jnp.int32)
        lo = jnp.bitwise_and(jnp.right_shift(b[:, 0::2], 16), 0xFFFF)
        hi = jnp.bitwise_and(b[:, 1::2], jnp.int32(-65536))
        return lo | hi

    layer = _make_layer(nblk)
    e1 = layer(_pack_bf16(ego0), packed, starts64, cnt).reshape(N_PAD, DIM)
    e2 = layer(_pack_bf16(e1), packed, starts64, cnt).reshape(N_PAD, DIM)
    e3 = layer(_pack_bf16(e2), packed, starts64, cnt).reshape(N_PAD, DIM)

    batch = users.shape[0]
    uidx = users.astype(jnp.int32)
    iidx = items.astype(jnp.int32) + nu
    final = _make_final(batch)
    u_out, i_out = final(ego0, e1, e2, e3, uidx, iidx)
    return (u_out, i_out)


# speedup vs baseline: 1.9756x; 1.9756x over previous
import functools

import jax
import jax.numpy as jnp
from jax import lax
from jax.experimental import pallas as pl
from jax.experimental.pallas import tpu as pltpu
from jax.experimental.pallas import tpu_sc as plsc

NC = 2
NS = 16
L = 16
NW = NC * NS
DIM = 64
ND = DIM // L
ROWS_W = 1568
N_PAD = ROWS_W * NW
C = 128
UNROLL = 8

_params = pltpu.CompilerParams(needs_layout_passes=False,
                               use_tc_tiling_on_sc=False)


def _worker_id():
    return lax.axis_index("s") * NC + lax.axis_index("c")


def _splat(ref, idx):
    return jnp.max(plsc.load_gather(ref, [jnp.full((L,), idx, jnp.int32)]))


def _make_compact(e_pad, nblk):
    mesh = plsc.VectorSubcoreMesh(core_axis_name="c", subcore_axis_name="s")

    @functools.partial(
        pl.kernel,
        mesh=mesh,
        compiler_params=_params,
        out_type=(jax.ShapeDtypeStruct((nblk, 3, C), jnp.int32),
                  jax.ShapeDtypeStruct((NW * L,), jnp.int32)),
        scratch_types=[
            pltpu.VMEM((64,), jnp.int32),
            pltpu.VMEM((C,), jnp.int32),
            pltpu.VMEM((C,), jnp.int32),
            pltpu.VMEM((C,), jnp.int32),
            pltpu.VMEM((C,), jnp.int32),
            pltpu.VMEM((C,), jnp.float32),
            pltpu.VMEM((C,), jnp.float32),
            pltpu.VMEM((2 * C,), jnp.int32),
            pltpu.VMEM((2 * C,), jnp.int32),
            pltpu.VMEM((2 * C,), jnp.int32),
            pltpu.VMEM((L,), jnp.int32),
            pltpu.SemaphoreType.DMA,
            pltpu.SemaphoreType.DMA,
        ],
    )
    def compact(row_hbm, col_hbm, val_hbm, starts_hbm, packed_hbm, cnt_hbm,
                starts_v, col_a, col_b, row_a, row_b, val_a, val_b,
                scol, sbase, sval, cnts, esem_a, esem_b):
        wid = _worker_id()
        r0 = wid * ROWS_W

        pltpu.sync_copy(starts_hbm, starts_v)
        start = _splat(starts_v, wid)
        end = _splat(starts_v, wid + 1)
        start_al = jnp.bitwise_and(start, jnp.int32(-8))
        nchunks = jnp.maximum((end - start_al + (C - 1)) // C, 1)
        npairs = (nchunks + 1) // 2
        b0 = start // C + 4 * wid

        bufs = ((col_a, row_a, val_a, esem_a),
                (col_b, row_b, val_b, esem_b))

        def _issue_edges(b, e):
            col, row, val, esem = bufs[b]
            pltpu.async_copy(col_hbm.at[pl.ds(e, C)], col, esem)
            pltpu.async_copy(row_hbm.at[pl.ds(e, C)], row, esem)
            pltpu.async_copy(val_hbm.at[pl.ds(e, C)], val, esem)

        def _wait_edges(b, e):
            col, row, val, esem = bufs[b]
            pltpu.make_async_copy(col_hbm.at[pl.ds(e, C)], col, esem).wait()
            pltpu.make_async_copy(row_hbm.at[pl.ds(e, C)], row, esem).wait()
            pltpu.make_async_copy(val_hbm.at[pl.ds(e, C)], val, esem).wait()

        def _flush(from_off, blk):
            pltpu.sync_copy(scol.at[pl.ds(from_off, C)],
                            packed_hbm.at[blk, 0])
            pltpu.sync_copy(sbase.at[pl.ds(from_off, C)],
                            packed_hbm.at[blk, 1])
            pltpu.sync_copy(sval.at[pl.ds(from_off, C)],
                            packed_hbm.at[blk, 2])

        _issue_edges(0, pl.multiple_of(start_al, 8))
        _issue_edges(1, pl.multiple_of(start_al + C, 8))

        def _pair(kk, carry):
            for b in (0, 1):
                k = kk * 2 + b
                col, row, val, _ = bufs[b]
                off, nb = carry
                _wait_edges(b, pl.multiple_of(start_al + k * C, 8))

                for i in range(C // L):
                    sl = pl.ds(i * L, L)
                    rl = row[sl] - r0
                    v = val[sl]
                    ok = (rl >= 0) & (rl < ROWS_W) & (v != 0.0)
                    basev = jnp.clip(rl, 0, ROWS_W - 1) * DIM
                    plsc.store_compressed(scol.at[pl.ds(off, L)],
                                          col[sl], mask=ok)
                    plsc.store_compressed(sbase.at[pl.ds(off, L)],
                                          basev, mask=ok)
                    plsc.store_compressed(sval.at[pl.ds(off, L)],
                                          plsc.bitcast(v, jnp.int32),
                                          mask=ok)
                    off = off + jnp.max(
                        plsc.all_reduce_population_count(ok))

                _issue_edges(b, pl.multiple_of(start_al + (k + 2) * C, 8))

                do_flush = off >= C

                @pl.when(do_flush)
                def _():
                    _flush(0, b0 + nb)
                    for i in range(C // L):
                        scol[pl.ds(i * L, L)] = scol[pl.ds(C + i * L, L)]
                        sbase[pl.ds(i * L, L)] = sbase[pl.ds(C + i * L, L)]
                        sval[pl.ds(i * L, L)] = sval[pl.ds(C + i * L, L)]

                off = jnp.where(do_flush, off - C, off)
                nb = jnp.where(do_flush, nb + 1, nb)
                carry = (off, nb)
            return carry

        off, nb = lax.fori_loop(0, npairs, _pair,
                                (jnp.int32(0), jnp.int32(0)))

        last = 2 * npairs
        _wait_edges(0, pl.multiple_of(start_al + last * C, 8))
        _wait_edges(1, pl.multiple_of(start_al + (last + 1) * C, 8))

        lane0 = lax.broadcasted_iota(jnp.int32, (L,), 0)
        zi = jnp.zeros((L,), jnp.int32)
        for i in range(2 * C // L):
            sl = pl.ds(i * L, L)
            m = (i * L + lane0) < off
            scol[sl] = jnp.where(m, scol[sl], zi)
            sbase[sl] = jnp.where(m, sbase[sl], zi)
            sval[sl] = jnp.where(m, sval[sl], zi)

        has_tail = off > 0

        @pl.when(has_tail)
        def _():
            _flush(0, b0 + nb)

        nb = jnp.where(has_tail, nb + 1, nb)
        odd = jnp.bitwise_and(nb, 1) == 1

        @pl.when(odd)
        def _():
            _flush(C, b0 + nb)

        nb = jnp.where(odd, nb + 1, nb)
        empty = nb == 0

        @pl.when(empty)
        def _():
            _flush(C, b0)
            _flush(C, b0 + 1)

        nb = jnp.where(empty, 2, nb)
        cnts[pl.ds(0, L)] = jnp.full((L,), 0, jnp.int32) + nb
        pltpu.sync_copy(cnts, cnt_hbm.at[pl.ds(wid * L, L)])

    return compact


def _make_layer(nblk):
    mesh = plsc.VectorSubcoreMesh(core_axis_name="c", subcore_axis_name="s")

    @functools.partial(
        pl.kernel,
        mesh=mesh,
        compiler_params=_params,
        out_type=jax.ShapeDtypeStruct((N_PAD * DIM,), jnp.float32),
        scratch_types=[
            pltpu.VMEM((64,), jnp.int32),
            pltpu.VMEM((NW * L,), jnp.int32),
            pltpu.VMEM((3, C), jnp.int32),
            pltpu.VMEM((3, C), jnp.int32),
            pltpu.VMEM((C, DIM // 2), jnp.int32),
            pltpu.VMEM((C, DIM // 2), jnp.int32),
            pltpu.VMEM((ROWS_W * DIM,), jnp.float32),
            pltpu.SemaphoreType.DMA,
            pltpu.SemaphoreType.DMA,
            pltpu.SemaphoreType.DMA,
            pltpu.SemaphoreType.DMA,
        ],
    )
    def layer(ego_hbm, packed_hbm, starts_hbm, cnt_hbm, out_hbm,
              starts_v, cnt_v, eb_a, eb_b, gath_a, gath_b, acc,
              esem_a, esem_b, gsem_a, gsem_b):
        wid = _worker_id()
        r0 = wid * ROWS_W

        pltpu.sync_copy(starts_hbm, starts_v)
        pltpu.sync_copy(cnt_hbm, cnt_v)
        start = _splat(starts_v, wid)
        b0 = start // C + 4 * wid
        nb = _splat(cnt_v, wid * L)
        npairs = nb // 2

        bufs = ((eb_a, gath_a, esem_a, gsem_a),
                (eb_b, gath_b, esem_b, gsem_b))

        def _issue_blk(b, k):
            eb, _, esem, _ = bufs[b]
            pltpu.async_copy(packed_hbm.at[b0 + k], eb, esem)

        def _wait_blk(b, k):
            eb, _, esem, _ = bufs[b]
            pltpu.make_async_copy(packed_hbm.at[b0 + k], eb, esem).wait()

        def _issue_gather(b):
            eb, gath, _, gsem = bufs[b]
            pltpu.async_copy(ego_hbm.at[eb.at[0]], gath, gsem)

        def _wait_gather(b):
            eb, gath, _, gsem = bufs[b]
            pltpu.make_async_copy(ego_hbm.at[eb.at[0]], gath, gsem).wait()

        _issue_blk(0, 0)
        _issue_blk(1, 1)

        zero = jnp.zeros((L,), jnp.float32)

        def _zero_rows(r, carry):
            for u in range(UNROLL):
                acc[pl.ds((r * UNROLL + u) * L, L)] = zero
            return carry

        lax.fori_loop(0, ROWS_W * DIM // (L * UNROLL), _zero_rows, 0)

        _wait_blk(0, 0)
        _issue_gather(0)

        dim_iota = [d * L + lax.broadcasted_iota(jnp.int32, (L,), 0)
                    for d in range(DIM // (2 * L))]
        one_v = jnp.full((L,), 1, jnp.int32)
        two_v = jnp.full((L,), 2, jnp.int32)

        def _pair(kk, carry):
            for b in (0, 1):
                k = kk * 2 + b
                eb, gath = bufs[b][0], bufs[b][1]
                nbuf = 1 - b

                _wait_gather(b)

                @pl.when(k + 1 < nb)
                def _():
                    _wait_blk(nbuf, k + 1)
                    _issue_gather(nbuf)

                @plsc.parallel_loop(0, C, 1, unroll=UNROLL)
                def _edges(c):
                    cv = jnp.full((L,), c, jnp.int32)
                    bv = plsc.load_gather(eb, [one_v, cv])
                    vv = plsc.bitcast(plsc.load_gather(eb, [two_v, cv]),
                                      jnp.float32)
                    for d in range(DIM // (2 * L)):
                        w = gath[c, pl.ds(d * L, L)]
                        fe = plsc.bitcast(jnp.left_shift(w, 16),
                                          jnp.float32)
                        fo = plsc.bitcast(
                            jnp.bitwise_and(w, jnp.int32(-65536)),
                            jnp.float32)
                        ide = bv + dim_iota[d]
                        plsc.addupdate_scatter(acc, [ide], fe * vv)
                        plsc.addupdate_scatter(acc, [ide + DIM // 2],
                                               fo * vv)

                _issue_blk(b, k + 2)
            return carry

        lax.fori_loop(0, npairs, _pair, 0)

        _wait_blk(0, nb)
        _wait_blk(1, nb + 1)

        pltpu.sync_copy(acc, out_hbm.at[pl.ds(r0 * DIM, ROWS_W * DIM)])

    return layer


def _make_final(batch):
    bpw = batch // NW
    mesh = plsc.VectorSubcoreMesh(core_axis_name="c", subcore_axis_name="s")
    out_sds = jax.ShapeDtypeStruct((batch, DIM), jnp.float32)

    @functools.partial(
        pl.kernel,
        mesh=mesh,
        compiler_params=_params,
        out_type=(out_sds, out_sds),
        scratch_types=[
            pltpu.VMEM((bpw,), jnp.int32),
            pltpu.VMEM((bpw, DIM), jnp.float32),
            pltpu.VMEM((bpw, DIM), jnp.float32),
            pltpu.VMEM((bpw, DIM), jnp.float32),
            pltpu.VMEM((bpw, DIM), jnp.float32),
            pltpu.VMEM((bpw, DIM), jnp.float32),
            pltpu.SemaphoreType.DMA,
        ],
    )
    def final(e0, e1, e2, e3, uidx_hbm, iidx_hbm, uout_hbm, iout_hbm,
              idxv, g0, g1, g2, g3, obuf, sem):
        wid = _worker_id()
        b0 = wid * bpw
        quarter = jnp.float32(0.25)
        for idx_hbm, out_hbm in ((uidx_hbm, uout_hbm), (iidx_hbm, iout_hbm)):
            pltpu.sync_copy(idx_hbm.at[pl.ds(b0, bpw)], idxv)
            for tab, gb in ((e0, g0), (e1, g1), (e2, g2), (e3, g3)):
                pltpu.async_copy(tab.at[idxv], gb, sem).wait()

            def _avg(r, carry):
                for d in range(ND):
                    sl = pl.ds(d * L, L)
                    obuf[r, sl] = (g0[r, sl] + g1[r, sl] + g2[r, sl]
                                   + g3[r, sl]) * quarter
                return carry

            lax.fori_loop(0, bpw, _avg, 0)
            pltpu.sync_copy(obuf, out_hbm.at[pl.ds(b0, bpw)])

    return final


def kernel(users, items, adj_row, adj_col, adj_val, user_emb, item_emb):
    nu, dim = user_emb.shape
    ni = item_emb.shape[0]
    n = nu + ni
    assert dim == DIM and n <= N_PAD
    e_cnt = adj_row.shape[0]

    mkey = jax.random.key(42)
    random_tensor = 0.5 + jax.random.uniform(mkey, adj_val.shape)
    mask = jnp.floor(random_tensor).astype(bool)
    vals = jnp.where(mask, adj_val, 0.0) * 2.0

    adj_row = adj_row.astype(jnp.int32)
    adj_col = adj_col.astype(jnp.int32)

    e_pad = ((e_cnt + C - 1) // C + 4) * C
    pe = e_pad - e_cnt
    rows_p = jnp.pad(adj_row, (0, pe))
    cols_p = jnp.pad(adj_col, (0, pe))
    vals_p = jnp.pad(vals, (0, pe))

    bounds = jnp.arange(NW + 1, dtype=jnp.int32) * ROWS_W
    starts = jnp.searchsorted(adj_row, bounds, side="left").astype(jnp.int32)
    starts64 = jnp.zeros((64,), jnp.int32).at[: NW + 1].set(starts)

    ego0 = jnp.pad(jnp.concatenate([user_emb, item_emb], axis=0),
                   ((0, N_PAD - n), (0, 0)))

    nblk = e_pad // C + 4 * NW + 8
    compact = _make_compact(e_pad, nblk)
    packed, cnt = compact(rows_p, cols_p, vals_p, starts64)

    def _pack_bf16(tbl):
        b = jax.lax.bitcast_convert_type(tbl, jnp.int32)
        lo = jnp.bitwise_and(jnp.right_shift(b[:, : DIM // 2], 16), 0xFFFF)
        hi = jnp.bitwise_and(b[:, DIM // 2 :], jnp.int32(-65536))
        return lo | hi

    layer = _make_layer(nblk)
    e1 = layer(_pack_bf16(ego0), packed, starts64, cnt).reshape(N_PAD, DIM)
    e2 = layer(_pack_bf16(e1), packed, starts64, cnt).reshape(N_PAD, DIM)
    e3 = layer(_pack_bf16(e2), packed, starts64, cnt).reshape(N_PAD, DIM)

    batch = users.shape[0]
    uidx = users.astype(jnp.int32)
    iidx = items.astype(jnp.int32) + nu
    final = _make_final(batch)
    u_out, i_out = final(ego0, e1, e2, e3, uidx, iidx)
    return (u_out, i_out)
